# baseline (speedup 1.0000x reference)
import functools

import jax
import jax.numpy as jnp
from jax import lax
from jax.experimental import pallas as pl
from jax.experimental.pallas import tpu as pltpu
from jax.experimental.pallas import tpu_sc as plsc

_FEATURE_SIZE = 1000000
_EMBED = 16
_BATCH = 16384
_FEATS = 26
_TOTAL = _BATCH * _FEATS

_NC, _NS = 2, 16
_NW = _NC * _NS
_CHUNK = 3328
_CPW = _TOTAL // (_NW * _CHUNK)


def _gather_body(idx_hbm, table_hbm, out_hbm, idx_v0, idx_v1, rows_v0,
                 rows_v1, sem0, sem1):
    wid = lax.axis_index("s") * _NC + lax.axis_index("c")
    idx_bufs = (idx_v0, idx_v1)
    row_bufs = (rows_v0, rows_v1)
    sems = (sem0, sem1)

    def chunk_base(j):
        return (wid * _CPW + j) * _CHUNK

    pltpu.sync_copy(idx_hbm.at[pl.ds(chunk_base(0), _CHUNK)], idx_bufs[0])
    pltpu.async_copy(table_hbm.at[idx_bufs[0]], row_bufs[0], sems[0])

    for j in range(_CPW):
        cur = j % 2
        nxt = (j + 1) % 2
        if j + 1 < _CPW:
            pltpu.sync_copy(
                idx_hbm.at[pl.ds(chunk_base(j + 1), _CHUNK)], idx_bufs[nxt]
            )
            pltpu.async_copy(
                table_hbm.at[idx_bufs[nxt]], row_bufs[nxt], sems[nxt]
            )
        pltpu.make_async_copy(
            table_hbm.at[idx_bufs[cur]], row_bufs[cur], sems[cur]
        ).wait()
        pltpu.sync_copy(
            row_bufs[cur], out_hbm.at[pl.ds(chunk_base(j), _CHUNK)]
        )


@jax.jit
def _gather(idx_flat, table):
    mesh = plsc.VectorSubcoreMesh(core_axis_name="c", subcore_axis_name="s")
    run = functools.partial(
        pl.kernel,
        mesh=mesh,
        out_type=jax.ShapeDtypeStruct((_TOTAL, _EMBED), jnp.float32),
        compiler_params=pltpu.CompilerParams(use_tc_tiling_on_sc=False),
        scratch_types=[
            pltpu.VMEM((_CHUNK,), jnp.int32),
            pltpu.VMEM((_CHUNK,), jnp.int32),
            pltpu.VMEM((_CHUNK, _EMBED), jnp.float32),
            pltpu.VMEM((_CHUNK, _EMBED), jnp.float32),
            pltpu.SemaphoreType.DMA,
            pltpu.SemaphoreType.DMA,
        ],
    )(_gather_body)
    return run(idx_flat, table)


def kernel(inputs, table):
    out_flat = _gather(inputs.reshape(_TOTAL), table)
    return out_flat.reshape(_BATCH, _FEATS, _EMBED)
